# baseline (speedup 1.0000x reference)
import functools

import jax
import jax.numpy as jnp
from jax import lax
from jax.experimental import pallas as pl
from jax.experimental.pallas import tpu as pltpu
from jax.experimental.pallas import tpu_sc as plsc

_B, _Q, _C = 64, 2048, 256
_R = _B * _Q
_NC, _NS = 2, 16
_NW = _NC * _NS
_RW = _R // _NW
_CH = 128
_NCHUNK = _RW // _CH
_GROUPS = _CH // 16
_UNROLL = 8


def _sc_body(lg_hbm, bx_hbm, lab_hbm, bout_hbm, cnt_hbm,
             lbuf, bxbuf, labbuf, boutbuf, cntbuf):
    cid = lax.axis_index("c")
    sid = lax.axis_index("s")
    wid = sid * _NC + cid
    base_row = wid * _RW

    iot = lax.iota(jnp.int32, 16)
    riot = lax.shift_right_logical(iot, 2)
    neg_inf = jnp.full((16,), -jnp.inf, jnp.float32)
    zero_f = jnp.zeros((16,), jnp.float32)
    zero_i = jnp.zeros((16,), jnp.int32)
    neg1 = jnp.full((16,), -1, jnp.int32)

    def chunk_body(ci, acc):
        row0 = base_row + ci * _CH
        pltpu.sync_copy(lg_hbm.at[pl.ds(row0 * _C, _CH * _C)], lbuf)
        pltpu.sync_copy(bx_hbm.at[pl.ds(row0 * 4, _CH * 4)], bxbuf)

        for g in range(_GROUPS):
            bvec = (g * 16 + iot) * _C

            def j_body(_, carry):
                best, bidxf, idxv = carry
                for _u in range(_UNROLL):
                    v = plsc.load_gather(lbuf, [idxv])
                    upd = v > best
                    best = jnp.maximum(best, v)
                    bidxf = jnp.where(upd, idxv, bidxf)
                    idxv = idxv + 1
                return best, bidxf, idxv

            best, bidxf, _ = lax.fori_loop(
                0, _C // _UNROLL, j_body, (neg_inf, bvec, bvec))

            cls = bidxf - bvec
            valid = best > zero_f
            labbuf[pl.ds(g * 16, 16)] = jnp.where(valid, cls, neg1)
            acc = acc + plsc.all_reduce_population_count(valid)

            for i in range(4):
                ridx = (g * 16 + 4 * i) + riot
                lv = plsc.load_gather(labbuf, [ridx])
                bx = bxbuf[pl.ds(g * 64 + i * 16, 16)]
                boutbuf[pl.ds(g * 64 + i * 16, 16)] = jnp.where(
                    lv >= zero_i, bx, zero_f)

        pltpu.sync_copy(labbuf, lab_hbm.at[pl.ds(row0, _CH)])
        pltpu.sync_copy(boutbuf, bout_hbm.at[pl.ds(row0 * 4, _CH * 4)])
        return acc

    acc = lax.fori_loop(0, _NCHUNK, chunk_body, jnp.zeros((16,), jnp.int32))
    cntbuf[...] = acc
    pltpu.sync_copy(cntbuf, cnt_hbm.at[wid])


_sc_call = functools.partial(
    pl.kernel,
    out_type=[
        jax.ShapeDtypeStruct((_R,), jnp.int32),
        jax.ShapeDtypeStruct((_R * 4,), jnp.float32),
        jax.ShapeDtypeStruct((_NW, 16), jnp.int32),
    ],
    mesh=plsc.VectorSubcoreMesh(core_axis_name="c", subcore_axis_name="s"),
    compiler_params=pltpu.CompilerParams(needs_layout_passes=False),
    scratch_types=[
        pltpu.VMEM((_CH * _C,), jnp.float32),
        pltpu.VMEM((_CH * 4,), jnp.float32),
        pltpu.VMEM((_CH,), jnp.int32),
        pltpu.VMEM((_CH * 4,), jnp.float32),
        pltpu.VMEM((16,), jnp.int32),
    ],
)(_sc_body)


@jax.jit
def kernel(pred_logits, pred_boxes):
    lab, bout, cnt = _sc_call(pred_logits.reshape(_R * _C),
                              pred_boxes.reshape(_R * 4))
    labels = lab.reshape(_B, _Q)
    boxes = bout.reshape(_B, _Q, 4)
    num_boxes = jnp.maximum(cnt[:, 0].sum().astype(jnp.float32), 1.0)
    return labels, boxes, num_boxes
